# Initial kernel scaffold; baseline (speedup 1.0000x reference)
import functools

import jax
import jax.numpy as jnp
from jax import lax
from jax.experimental import pallas as pl
from jax.experimental.pallas import tpu as pltpu
from jax.experimental.pallas import tpu_sc as plsc

N_NODE = 10000
E = 320000
D_IN = 128
D_H = 16

NC = 2
NS = 16
NW = NC * NS
PER_W = E // NW
CH = 80
NCHUNK = PER_W // CH

_HIGH = jax.lax.Precision.HIGHEST


def _node_body(var_ref, con_ref, vW1r, vb1r, vW2r, vb2r, cW1r, cb1r, cW2r,
               cb2r, pVr, pCr, tv_ref, tc_ref):
    v = jnp.maximum(jnp.dot(var_ref[:], vW1r[:], precision=_HIGH) + vb1r[:], 0.0)
    v = jnp.maximum(jnp.dot(v, vW2r[:], precision=_HIGH) + vb2r[:], 0.0)
    tv_ref[:] = jnp.dot(v, pVr[:], precision=_HIGH)
    c = jnp.maximum(jnp.dot(con_ref[:], cW1r[:], precision=_HIGH) + cb1r[:], 0.0)
    c = jnp.maximum(jnp.dot(c, cW2r[:], precision=_HIGH) + cb2r[:], 0.0)
    tc_ref[:] = jnp.dot(c, pCr[:], precision=_HIGH)


def _edge_body(cef_ref, gv_ref, gc_ref, w1r, b1r, w2r, b2r, out_ref):
    h = jnp.dot(cef_ref[:], w1r[:], precision=_HIGH) + gv_ref[:] + gc_ref[:] + b1r[:]
    h = jnp.maximum(h, 0.0)
    out_ref[:] = jnp.dot(h, w2r[:], precision=_HIGH) + b2r[:]


def _gather_body(tv_hbm, tc_hbm, ei_hbm, gv_hbm, gc_hbm,
                 idx_v, idx_c, rows_v, rows_c, sem_v, sem_c):
    wid = lax.axis_index("s") * NC + lax.axis_index("c")
    base = wid * PER_W
    pltpu.sync_copy(ei_hbm.at[0, wid], idx_v)
    pltpu.sync_copy(ei_hbm.at[1, wid], idx_c)

    def body(j, carry):
        row0 = base + j * CH
        cpv = pltpu.make_async_copy(tv_hbm.at[idx_v.at[j]], rows_v, sem_v)
        cpc = pltpu.make_async_copy(tc_hbm.at[idx_c.at[j]], rows_c, sem_c)
        cpv.start()
        cpc.start()
        cpv.wait()
        cpc.wait()
        pltpu.sync_copy(rows_v, gv_hbm.at[pl.ds(row0, CH), :])
        pltpu.sync_copy(rows_c, gc_hbm.at[pl.ds(row0, CH), :])
        return carry

    lax.fori_loop(0, NCHUNK, body, 0)


_sc_gather = functools.partial(
    pl.kernel,
    out_type=(
        jax.ShapeDtypeStruct((E, D_H), jnp.float32),
        jax.ShapeDtypeStruct((E, D_H), jnp.float32),
    ),
    mesh=plsc.VectorSubcoreMesh(core_axis_name="c", subcore_axis_name="s"),
    scratch_types=[
        pltpu.VMEM((NCHUNK, CH), jnp.int32),
        pltpu.VMEM((NCHUNK, CH), jnp.int32),
        pltpu.VMEM((CH, D_H), jnp.float32),
        pltpu.VMEM((CH, D_H), jnp.float32),
        pltpu.SemaphoreType.DMA,
        pltpu.SemaphoreType.DMA,
    ],
)(_gather_body)


def kernel(var_f, con_f, combined_edge_f, edge_index_var_con,
           vW1, vb1, vW2, vb2, cW1, cb1, cW2, cb2, eW1, eb1, eW2, eb2):
    w1e = eW1[:D_H]
    pV = eW1[D_H:2 * D_H]
    pC = eW1[2 * D_H:]

    tv, tc = pl.pallas_call(
        _node_body,
        out_shape=(
            jax.ShapeDtypeStruct((N_NODE, D_H), jnp.float32),
            jax.ShapeDtypeStruct((N_NODE, D_H), jnp.float32),
        ),
    )(var_f, con_f, vW1, vb1.reshape(1, D_H), vW2, vb2.reshape(1, D_H),
      cW1, cb1.reshape(1, D_H), cW2, cb2.reshape(1, D_H), pV, pC)

    ei = edge_index_var_con.astype(jnp.int32).reshape(2, NW, NCHUNK, CH)
    gv, gc = _sc_gather(tv, tc, ei)

    eb = E // 100
    out = pl.pallas_call(
        _edge_body,
        grid=(E // eb,),
        in_specs=[
            pl.BlockSpec((eb, D_H), lambda i: (i, 0)),
            pl.BlockSpec((eb, D_H), lambda i: (i, 0)),
            pl.BlockSpec((eb, D_H), lambda i: (i, 0)),
            pl.BlockSpec((D_H, D_H), lambda i: (0, 0)),
            pl.BlockSpec((1, D_H), lambda i: (0, 0)),
            pl.BlockSpec((D_H, D_H), lambda i: (0, 0)),
            pl.BlockSpec((1, D_H), lambda i: (0, 0)),
        ],
        out_specs=pl.BlockSpec((eb, D_H), lambda i: (i, 0)),
        out_shape=jax.ShapeDtypeStruct((E, D_H), jnp.float32),
    )(combined_edge_f, gv, gc, w1e, eb1.reshape(1, D_H), eW2,
      eb2.reshape(1, D_H))
    return out


# speedup vs baseline: 1.5619x; 1.5619x over previous
import functools

import jax
import jax.numpy as jnp
from jax import lax
from jax.experimental import pallas as pl
from jax.experimental.pallas import tpu as pltpu
from jax.experimental.pallas import tpu_sc as plsc

N_NODE = 10000
E = 320000
D_IN = 128
D_H = 16

NC = 2
NS = 16
NW = NC * NS
PER_W = E // NW
CH = 80
NCHUNK = PER_W // CH

_HIGH = jax.lax.Precision.HIGHEST


def _node_body(var_ref, con_ref, vW1r, vb1r, vW2r, vb2r, cW1r, cb1r, cW2r,
               cb2r, pVr, pCr, tv_ref, tc_ref):
    v = jnp.maximum(jnp.dot(var_ref[:], vW1r[:], precision=_HIGH) + vb1r[:], 0.0)
    v = jnp.maximum(jnp.dot(v, vW2r[:], precision=_HIGH) + vb2r[:], 0.0)
    tv_ref[:] = jnp.dot(v, pVr[:], precision=_HIGH)
    c = jnp.maximum(jnp.dot(con_ref[:], cW1r[:], precision=_HIGH) + cb1r[:], 0.0)
    c = jnp.maximum(jnp.dot(c, cW2r[:], precision=_HIGH) + cb2r[:], 0.0)
    tc_ref[:] = jnp.dot(c, pCr[:], precision=_HIGH)


def _edge_body(cef_ref, gv_ref, gc_ref, w1r, b1r, w2r, b2r, out_ref):
    h = jnp.dot(cef_ref[:], w1r[:], precision=_HIGH) + gv_ref[:] + gc_ref[:] + b1r[:]
    h = jnp.maximum(h, 0.0)
    out_ref[:] = jnp.dot(h, w2r[:], precision=_HIGH) + b2r[:]


def _gather_body(tv_hbm, tc_hbm, ei_hbm, gv_hbm, gc_hbm,
                 idx_v, idx_c, rows_v, rows_c, sem_v, sem_c):
    wid = lax.axis_index("s") * NC + lax.axis_index("c")
    base = wid * PER_W
    pltpu.sync_copy(ei_hbm.at[0, wid], idx_v)
    pltpu.sync_copy(ei_hbm.at[1, wid], idx_c)

    def body(j, carry):
        row0 = base + j * CH
        cpv = pltpu.make_async_copy(tv_hbm.at[idx_v.at[j]], rows_v, sem_v)
        cpc = pltpu.make_async_copy(tc_hbm.at[idx_c.at[j]], rows_c, sem_c)
        cpv.start()
        cpc.start()
        cpv.wait()
        cpc.wait()
        pltpu.sync_copy(rows_v, gv_hbm.at[pl.ds(row0, CH), :])
        pltpu.sync_copy(rows_c, gc_hbm.at[pl.ds(row0, CH), :])
        return carry

    lax.fori_loop(0, NCHUNK, body, 0)


_sc_gather = functools.partial(
    pl.kernel,
    out_type=(
        jax.ShapeDtypeStruct((E, D_H), jnp.float32),
        jax.ShapeDtypeStruct((E, D_H), jnp.float32),
    ),
    mesh=plsc.VectorSubcoreMesh(core_axis_name="c", subcore_axis_name="s"),
    scratch_types=[
        pltpu.VMEM((NCHUNK, CH), jnp.int32),
        pltpu.VMEM((NCHUNK, CH), jnp.int32),
        pltpu.VMEM((CH, D_H), jnp.float32),
        pltpu.VMEM((CH, D_H), jnp.float32),
        pltpu.SemaphoreType.DMA,
        pltpu.SemaphoreType.DMA,
    ],
    compiler_params=pltpu.CompilerParams(use_tc_tiling_on_sc=False),
)(_gather_body)


def kernel(var_f, con_f, combined_edge_f, edge_index_var_con,
           vW1, vb1, vW2, vb2, cW1, cb1, cW2, cb2, eW1, eb1, eW2, eb2):
    w1e = eW1[:D_H]
    pV = eW1[D_H:2 * D_H]
    pC = eW1[2 * D_H:]

    tv, tc = pl.pallas_call(
        _node_body,
        out_shape=(
            jax.ShapeDtypeStruct((N_NODE, D_H), jnp.float32),
            jax.ShapeDtypeStruct((N_NODE, D_H), jnp.float32),
        ),
    )(var_f, con_f, vW1, vb1.reshape(1, D_H), vW2, vb2.reshape(1, D_H),
      cW1, cb1.reshape(1, D_H), cW2, cb2.reshape(1, D_H), pV, pC)

    ei = edge_index_var_con.astype(jnp.int32).reshape(2, NW, NCHUNK, CH)
    gv, gc = _sc_gather(tv, tc, ei)

    eb = E // 100
    out = pl.pallas_call(
        _edge_body,
        grid=(E // eb,),
        in_specs=[
            pl.BlockSpec((eb, D_H), lambda i: (i, 0)),
            pl.BlockSpec((eb, D_H), lambda i: (i, 0)),
            pl.BlockSpec((eb, D_H), lambda i: (i, 0)),
            pl.BlockSpec((D_H, D_H), lambda i: (0, 0)),
            pl.BlockSpec((1, D_H), lambda i: (0, 0)),
            pl.BlockSpec((D_H, D_H), lambda i: (0, 0)),
            pl.BlockSpec((1, D_H), lambda i: (0, 0)),
        ],
        out_specs=pl.BlockSpec((eb, D_H), lambda i: (i, 0)),
        out_shape=jax.ShapeDtypeStruct((E, D_H), jnp.float32),
    )(combined_edge_f, gv, gc, w1e, eb1.reshape(1, D_H), eW2,
      eb2.reshape(1, D_H))
    return out


# speedup vs baseline: 3.8522x; 2.4664x over previous
import functools

import jax
import jax.numpy as jnp
from jax import lax
from jax.experimental import pallas as pl
from jax.experimental.pallas import tpu as pltpu
from jax.experimental.pallas import tpu_sc as plsc

N_NODE = 10000
E = 320000
D_IN = 128
D_H = 16

NC = 2
NS = 16
NW = NC * NS
PER_W = E // NW
CH = 80
NCHUNK = PER_W // CH
GRP = 5
NGRP = NCHUNK // GRP
GROWS = GRP * CH

_HIGH = jax.lax.Precision.HIGHEST


def _node_body(var_ref, con_ref, w1v, w1c, b1, w2, b2, w3, tv_ref, tc_ref):
    t = (jnp.dot(var_ref[:], w1v[:], precision=_HIGH)
         + jnp.dot(con_ref[:], w1c[:], precision=_HIGH) + b1[:])
    t = jnp.maximum(t, 0.0)
    t = jnp.maximum(jnp.dot(t, w2[:], precision=_HIGH) + b2[:], 0.0)
    t = jnp.dot(t, w3[:], precision=_HIGH)
    tv_ref[:] = t[:, :D_H]
    tc_ref[:] = t[:, D_H:]


def _edge_body(cef_ref, gv_ref, gc_ref, w1r, b1r, w2r, b2r, out_ref):
    h = jnp.dot(cef_ref[:], w1r[:], precision=_HIGH) + gv_ref[:] + gc_ref[:] + b1r[:]
    h = jnp.maximum(h, 0.0)
    out_ref[:] = jnp.dot(h, w2r[:], precision=_HIGH) + b2r[:]


def _gather_body(tv_hbm, tc_hbm, ei_hbm, gv_hbm, gc_hbm,
                 idx_v, idx_c, rows_v, rows_c, sem_v, sem_c):
    wid = lax.axis_index("s") * NC + lax.axis_index("c")
    base = wid * PER_W
    pltpu.sync_copy(ei_hbm.at[0, wid], idx_v)
    pltpu.sync_copy(ei_hbm.at[1, wid], idx_c)

    def fire(g, p):
        handles = []
        for k in range(GRP):
            j = g * GRP + k
            cpv = pltpu.make_async_copy(
                tv_hbm.at[idx_v.at[j]],
                rows_v.at[p, pl.ds(k * CH, CH), :], sem_v)
            cpc = pltpu.make_async_copy(
                tc_hbm.at[idx_c.at[j]],
                rows_c.at[p, pl.ds(k * CH, CH), :], sem_c)
            cpv.start()
            cpc.start()
            handles.append((cpv, cpc))
        return handles

    def drain(g, p, handles):
        for cpv, cpc in handles:
            cpv.wait()
            cpc.wait()
        row0 = base + g * GROWS
        pltpu.sync_copy(rows_v.at[p], gv_hbm.at[pl.ds(row0, GROWS), :])
        pltpu.sync_copy(rows_c.at[p], gc_hbm.at[pl.ds(row0, GROWS), :])

    prev = fire(0, 0)
    for g in range(1, NGRP):
        cur = fire(g, g % 2)
        drain(g - 1, (g - 1) % 2, prev)
        prev = cur
    drain(NGRP - 1, (NGRP - 1) % 2, prev)


_sc_gather = functools.partial(
    pl.kernel,
    out_type=(
        jax.ShapeDtypeStruct((E, D_H), jnp.float32),
        jax.ShapeDtypeStruct((E, D_H), jnp.float32),
    ),
    mesh=plsc.VectorSubcoreMesh(core_axis_name="c", subcore_axis_name="s"),
    scratch_types=[
        pltpu.VMEM((NCHUNK, CH), jnp.int32),
        pltpu.VMEM((NCHUNK, CH), jnp.int32),
        pltpu.VMEM((2, GROWS, D_H), jnp.float32),
        pltpu.VMEM((2, GROWS, D_H), jnp.float32),
        pltpu.SemaphoreType.DMA,
        pltpu.SemaphoreType.DMA,
    ],
    compiler_params=pltpu.CompilerParams(use_tc_tiling_on_sc=False),
)(_gather_body)


def kernel(var_f, con_f, combined_edge_f, edge_index_var_con,
           vW1, vb1, vW2, vb2, cW1, cb1, cW2, cb2, eW1, eb1, eW2, eb2):
    f32 = jnp.float32
    z = jnp.zeros((D_IN, D_H), f32)
    w1v = jnp.concatenate([vW1, z], axis=1)
    w1c = jnp.concatenate([z, cW1], axis=1)
    b1 = jnp.concatenate([vb1, cb1]).reshape(1, 2 * D_H)
    z2 = jnp.zeros((D_H, D_H), f32)
    w2 = jnp.block([[vW2, z2], [z2, cW2]])
    b2 = jnp.concatenate([vb2, cb2]).reshape(1, 2 * D_H)
    w3 = jnp.block([[eW1[D_H:2 * D_H], z2], [z2, eW1[2 * D_H:]]])

    tv, tc = pl.pallas_call(
        _node_body,
        out_shape=(
            jax.ShapeDtypeStruct((N_NODE, D_H), f32),
            jax.ShapeDtypeStruct((N_NODE, D_H), f32),
        ),
    )(var_f, con_f, w1v, w1c, b1, w2, b2, w3)

    ei = edge_index_var_con.astype(jnp.int32).reshape(2, NW, NCHUNK, CH)
    gv, gc = _sc_gather(tv, tc, ei)

    eye8 = jnp.eye(8, dtype=f32)
    w1bd = jnp.kron(eye8, eW1[:D_H])
    w2bd = jnp.kron(eye8, eW2)
    b1t = jnp.tile(eb1, 8).reshape(1, D_IN)
    b2t = jnp.tile(eb2, 8).reshape(1, D_IN)
    ER = E // 8
    cef_r = combined_edge_f.reshape(ER, D_IN)
    gv_r = gv.reshape(ER, D_IN)
    gc_r = gc.reshape(ER, D_IN)

    eb = ER // 20
    out = pl.pallas_call(
        _edge_body,
        grid=(ER // eb,),
        in_specs=[
            pl.BlockSpec((eb, D_IN), lambda i: (i, 0)),
            pl.BlockSpec((eb, D_IN), lambda i: (i, 0)),
            pl.BlockSpec((eb, D_IN), lambda i: (i, 0)),
            pl.BlockSpec((D_IN, D_IN), lambda i: (0, 0)),
            pl.BlockSpec((1, D_IN), lambda i: (0, 0)),
            pl.BlockSpec((D_IN, D_IN), lambda i: (0, 0)),
            pl.BlockSpec((1, D_IN), lambda i: (0, 0)),
        ],
        out_specs=pl.BlockSpec((eb, D_IN), lambda i: (i, 0)),
        out_shape=jax.ShapeDtypeStruct((ER, D_IN), f32),
    )(cef_r, gv_r, gc_r, w1bd, b1t, w2bd, b2t)
    return out.reshape(E, D_H)


# speedup vs baseline: 5.9073x; 1.5335x over previous
import functools

import jax
import jax.numpy as jnp
from jax import lax
from jax.experimental import pallas as pl
from jax.experimental.pallas import tpu as pltpu
from jax.experimental.pallas import tpu_sc as plsc

N_NODE = 10000
E = 320000
D_IN = 128
D_H = 16

NC = 2
NS = 16
NW = NC * NS
NPANEL = E // 128
NPW = NPANEL // NW
NEXTRA = NPANEL - NPW * NW

_HIGH = jax.lax.Precision.HIGHEST


def _node_body(var_ref, con_ref, w1v, w1c, b1, w2, b2, w3, tv_ref, tc_ref):
    t = (jnp.dot(var_ref[:], w1v[:], precision=_HIGH)
         + jnp.dot(con_ref[:], w1c[:], precision=_HIGH) + b1[:])
    t = jnp.maximum(t, 0.0)
    t = jnp.maximum(jnp.dot(t, w2[:], precision=_HIGH) + b2[:], 0.0)
    t = jnp.dot(t, w3[:], precision=_HIGH)
    tv_ref[:] = t[:, :D_H]
    tc_ref[:] = t[:, D_H:]


def _edge_body(g4_ref, cef_ref, w1t_ref, b1_ref, w2t_ref, b2_ref, out_ref):
    pb = g4_ref.shape[1]
    n = 128 * pb
    gu = jnp.transpose(g4_ref[0], (1, 0, 2)).reshape(8, n)
    gl = jnp.transpose(g4_ref[1], (1, 0, 2)).reshape(8, n)
    g = jnp.concatenate([gu, gl], axis=0)
    x = jnp.dot(w1t_ref[:], cef_ref[:], precision=_HIGH) + g + b1_ref[:, 0:1]
    x = jnp.maximum(x, 0.0)
    out_ref[:] = jnp.dot(w2t_ref[:], x, precision=_HIGH) + b2_ref[:, 0:1]


def _gather_body(tv_hbm, tc_hbm, ei_hbm, g4_hbm,
                 idx_v, idx_c, idx_xv, idx_xc, rows_v, rows_c, stage,
                 sem_v, sem_c):
    wid = lax.axis_index("s") * NC + lax.axis_index("c")
    p0 = wid * NPW
    pltpu.sync_copy(ei_hbm.at[0, pl.ds(p0, NPW)], idx_v)
    pltpu.sync_copy(ei_hbm.at[1, pl.ds(p0, NPW)], idx_c)

    iota16 = lax.iota(jnp.int32, 16)

    def fire(idxv_row, idxc_row, slot):
        pltpu.make_async_copy(tv_hbm.at[idxv_row], rows_v.at[slot],
                              sem_v).start()
        pltpu.make_async_copy(tc_hbm.at[idxc_row], rows_c.at[slot],
                              sem_c).start()

    def wait(idxv_row, idxc_row, slot):
        pltpu.make_async_copy(tv_hbm.at[idxv_row], rows_v.at[slot],
                              sem_v).wait()
        pltpu.make_async_copy(tc_hbm.at[idxc_row], rows_c.at[slot],
                              sem_c).wait()

    def transpose_panel(slot):
        def col_body(col, _):
            vv = rows_v[slot, col, :]
            vc = rows_c[slot, col, :]
            plsc.store_scatter(stage, [iota16, jnp.full((16,), col, jnp.int32)],
                               vv + vc)
            return 0
        lax.fori_loop(0, 128, col_body, 0, unroll=4)

    def write_panel(panel):
        pltpu.sync_copy(stage.at[pl.ds(0, 8), :], g4_hbm.at[0, panel])
        pltpu.sync_copy(stage.at[pl.ds(8, 8), :], g4_hbm.at[1, panel])

    fire(idx_v.at[0], idx_c.at[0], 0)

    def body(p, _):
        slot = lax.rem(p, 2)
        nslot = lax.rem(p + 1, 2)

        @pl.when(p + 1 < NPW)
        def _():
            fire(idx_v.at[p + 1], idx_c.at[p + 1], nslot)

        wait(idx_v.at[p], idx_c.at[p], slot)
        transpose_panel(slot)
        write_panel(p0 + p)
        return 0

    lax.fori_loop(0, NPW, body, 0)

    @pl.when(wid < NEXTRA)
    def _():
        xp = NW * NPW + wid
        pltpu.sync_copy(ei_hbm.at[0, pl.ds(xp, 1)], idx_xv)
        pltpu.sync_copy(ei_hbm.at[1, pl.ds(xp, 1)], idx_xc)
        fire(idx_xv.at[0], idx_xc.at[0], 0)
        wait(idx_xv.at[0], idx_xc.at[0], 0)
        transpose_panel(0)
        write_panel(xp)


_sc_gather = functools.partial(
    pl.kernel,
    out_type=jax.ShapeDtypeStruct((2, NPANEL, 8, 128), jnp.float32),
    mesh=plsc.VectorSubcoreMesh(core_axis_name="c", subcore_axis_name="s"),
    scratch_types=[
        pltpu.VMEM((NPW, 128), jnp.int32),
        pltpu.VMEM((NPW, 128), jnp.int32),
        pltpu.VMEM((1, 128), jnp.int32),
        pltpu.VMEM((1, 128), jnp.int32),
        pltpu.VMEM((2, 128, D_H), jnp.float32),
        pltpu.VMEM((2, 128, D_H), jnp.float32),
        pltpu.VMEM((16, 128), jnp.float32),
        pltpu.SemaphoreType.DMA,
        pltpu.SemaphoreType.DMA,
    ],
    compiler_params=pltpu.CompilerParams(use_tc_tiling_on_sc=False,
                                         needs_layout_passes=False),
)(_gather_body)


def kernel(var_f, con_f, combined_edge_f, edge_index_var_con,
           vW1, vb1, vW2, vb2, cW1, cb1, cW2, cb2, eW1, eb1, eW2, eb2):
    f32 = jnp.float32
    z = jnp.zeros((D_IN, D_H), f32)
    w1v = jnp.concatenate([vW1, z], axis=1)
    w1c = jnp.concatenate([z, cW1], axis=1)
    b1 = jnp.concatenate([vb1, cb1]).reshape(1, 2 * D_H)
    z2 = jnp.zeros((D_H, D_H), f32)
    w2 = jnp.block([[vW2, z2], [z2, cW2]])
    b2 = jnp.concatenate([vb2, cb2]).reshape(1, 2 * D_H)
    w3 = jnp.block([[eW1[D_H:2 * D_H], z2], [z2, eW1[2 * D_H:]]])

    tv, tc = pl.pallas_call(
        _node_body,
        out_shape=(
            jax.ShapeDtypeStruct((N_NODE, D_H), f32),
            jax.ShapeDtypeStruct((N_NODE, D_H), f32),
        ),
    )(var_f, con_f, w1v, w1c, b1, w2, b2, w3)

    ei = edge_index_var_con.astype(jnp.int32).reshape(2, NPANEL, 128)
    g4 = _sc_gather(tv, tc, ei)

    cefT = combined_edge_f.T
    w1t = eW1[:D_H].T
    w2t = eW2.T
    b1e = jnp.tile(eb1.reshape(D_H, 1), (1, 128))
    b2e = jnp.tile(eb2.reshape(D_H, 1), (1, 128))

    PB = 125
    outT = pl.pallas_call(
        _edge_body,
        grid=(NPANEL // PB,),
        in_specs=[
            pl.BlockSpec((2, PB, 8, 128), lambda i: (0, i, 0, 0)),
            pl.BlockSpec((D_H, 128 * PB), lambda i: (0, i)),
            pl.BlockSpec((D_H, D_H), lambda i: (0, 0)),
            pl.BlockSpec((D_H, 128), lambda i: (0, 0)),
            pl.BlockSpec((D_H, D_H), lambda i: (0, 0)),
            pl.BlockSpec((D_H, 128), lambda i: (0, 0)),
        ],
        out_specs=pl.BlockSpec((D_H, 128 * PB), lambda i: (0, i)),
        out_shape=jax.ShapeDtypeStruct((D_H, E), f32),
    )(g4, cefT, w1t, b1e, w2t, b2e)
    return outT.T
